# baseline (speedup 1.0000x reference)
import jax
import jax.numpy as jnp
from jax import lax
from jax.experimental import pallas as pl
from jax.experimental.pallas import tpu as pltpu
from jax.experimental.pallas import tpu_sc as plsc

N = 524288
C = 64
M = 524288
NC = 2
NS = 16
NW = NC * NS
P = N // NW
L = 16
PB = 512
OB = 256
TPB = PB // 128
TOB = OB // 128
IDW = PB + 32
NTI = N // 128
NTO = M // 128


def _body(d4, seg, z4, o4, ids_w, bpos, bid, in_v, ov0, ov1, ov2,
          sem_i0, sem_i1, sem_0, sem_1, sem_2,
          sem_z0, sem_z1, sem_z2, sem_t):
    wid = lax.axis_index("s") * NC + lax.axis_index("c")
    iota = lax.broadcasted_iota(jnp.int32, (L,), 0)
    ones = jnp.full((L,), 1, jnp.int32)
    zeros = jnp.full((L,), 0, jnp.int32)
    zf16 = jnp.zeros((L,), jnp.float32)

    pltpu.sync_copy(seg.at[pl.ds(wid * P, L)], ids_w.at[pl.ds(0, L)])

    @pl.when(wid < NW - 1)
    def _():
        pltpu.sync_copy(seg.at[pl.ds((wid + 1) * P, L)], ids_w.at[pl.ds(L, L)])

    myfirst = ids_w[pl.ds(0, L)][0]
    nxtfirst = ids_w[pl.ds(L, L)][0]
    Tw = jnp.where(wid == 0, 0, myfirst & -128)
    Tn = jnp.where(wid == NW - 1, M, nxtfirst & -128)

    def find_first_ge(Tv, anchor):
        tsplat = jnp.full((L,), 0, jnp.int32) + Tv

        def count_win(e):
            pltpu.sync_copy(seg.at[pl.ds(pl.multiple_of(e - 512, 512), 512)],
                            ids_w.at[pl.ds(0, 512)])

            def cg(g, acc):
                v = ids_w[pl.ds(g * L, L)]
                return acc + plsc.all_reduce_population_count(v >= tsplat)

            return lax.fori_loop(0, 512 // L, cg, zeros)[0]

        c0 = count_win(anchor)

        def cond(st):
            e, c = st
            return jnp.logical_and(c == 512, e > 512)

        def bdy(st):
            e, c = st
            return (e - 512, count_win(e - 512))

        eF, cF = lax.while_loop(cond, bdy, (anchor, c0))
        return eF - cF

    lo = jnp.where(wid == 0, 0,
                   find_first_ge(Tw, jnp.maximum(wid * P, 512)))
    hi = jnp.where(wid == NW - 1, N, find_first_ge(Tn, (wid + 1) * P))

    def zfill_vec(ov):
        def zz(g, c):
            for t in range(TOB):
                for c8 in range(8):
                    for q in range(8):
                        ov[g, t, c8, pl.ds(q * L, L)] = zf16
            return c

        lax.fori_loop(0, 8, zz, 0)

    def fire_flush(ov, sem, S):
        for g8 in range(8):
            pltpu.async_copy(ov.at[g8], o4.at[g8, pl.ds(S >> 7, TOB)], sem)

    def wait_flush(ov, sem):
        for g8 in range(8):
            pltpu.make_async_copy(ov.at[g8], o4.at[g8, pl.ds(0, TOB)],
                                  sem).wait()

    def fire_zero(ov, sem):
        for g8 in range(8):
            pltpu.async_copy(z4.at[g8], ov.at[g8], sem)

    def wait_zero(ov, sem):
        for g8 in range(8):
            pltpu.make_async_copy(z4.at[g8], ov.at[g8], sem).wait()

    def on_slot(s, fn):
        def b0(c):
            fn(ov0, sem_0, sem_z0)
            return c

        def b1(c):
            fn(ov1, sem_1, sem_z1)
            return c

        def b2(c):
            fn(ov2, sem_2, sem_z2)
            return c

        def rest(c):
            return lax.cond(s == 1, b1, b2, c)

        lax.cond(s == 0, b0, rest, 0)

    def flush_step(S, slot, fl, zf):
        B = jnp.where(slot == 2, 0, slot + 1)
        C = jnp.where(B == 2, 0, B + 1)
        on_slot(slot, lambda ov, fs, zs: fire_flush(ov, fs, S))

        zfB = (zf >> B) & 1

        def activate(ov, fs, zs):
            @pl.when(zfB == 1)
            def _():
                wait_zero(ov, zs)

        on_slot(B, activate)

        flC = (fl >> C) & 1

        def prep(ov, fs, zs):
            @pl.when(flC == 1)
            def _():
                wait_flush(ov, fs)

            fire_zero(ov, zs)

        on_slot(C, prep)
        one = jnp.int32(1)
        nfl = (fl | (one << slot)) & (7 ^ (one << C))
        nzf = (zf & (7 ^ (one << B))) | (one << C)
        return S + OB, B, nfl, nzf

    def binit(g, c):
        bid[pl.ds(g * L, L)] = zeros
        bpos[pl.ds(g * L, L)] = zeros
        return c

    lax.fori_loop(0, (IDW + L) // L, binit, 0)
    zfill_vec(ov0)
    fire_zero(ov1, sem_z1)
    pb0 = lo & -128
    nblk = jnp.where(hi > pb0, (hi - pb0 + PB - 1) // PB, 0)

    def place_16(ov, slv, posv, idv, S, mask):
        ti = posv >> 7
        li = posv & 127
        to = (idv - S) >> 7
        lo_ = (idv - S) & 127
        for g8 in range(8):
            sg = jnp.full((L,), g8, jnp.int32)
            vals = []
            for c8 in range(8):
                sc = jnp.full((L,), c8, jnp.int32)
                vals.append(
                    plsc.load_gather(in_v, [slv, sg, ti, sc, li], mask=mask))
            for c8 in range(8):
                sc = jnp.full((L,), c8, jnp.int32)
                plsc.store_scatter(ov, [sg, to, sc, lo_], vals[c8], mask=mask)

    def fire_input(b):
        pbs = jnp.minimum(pb0 + b * PB, N - PB)
        par = b & 1

        @pl.when(par == 0)
        def _():
            for g8 in range(8):
                pltpu.async_copy(d4.at[g8, pl.ds(pbs >> 7, TPB)],
                                 in_v.at[0, g8], sem_i0)

        @pl.when(par == 1)
        def _():
            for g8 in range(8):
                pltpu.async_copy(d4.at[g8, pl.ds(pbs >> 7, TPB)],
                                 in_v.at[1, g8], sem_i1)

    def wait_input(b):
        par = b & 1

        @pl.when(par == 0)
        def _():
            for g8 in range(8):
                pltpu.make_async_copy(d4.at[0, pl.ds(0, TPB)],
                                      in_v.at[0, g8], sem_i0).wait()

        @pl.when(par == 1)
        def _():
            for g8 in range(8):
                pltpu.make_async_copy(d4.at[0, pl.ds(0, TPB)],
                                      in_v.at[1, g8], sem_i1).wait()

    def blk_body(b, st):
        S, slot, fl, zf = st
        pb = pb0 + b * PB
        pbs = jnp.minimum(pb, N - PB)
        pmin = jnp.maximum(pb, lo)
        pe = jnp.minimum(pb + PB, hi)

        fire_input(b + 1)

        as_ = jnp.minimum(pb, N - (PB + 16))
        pltpu.sync_copy(seg.at[pl.ds(pl.multiple_of(as_, 16), PB + 16)],
                        ids_w.at[pl.ds(0, PB + 16)])

        @pl.when(as_ == N - (PB + 16))
        def _():
            ids_w[pl.ds(PB + 16, L)] = jnp.full((L,), M, jnp.int32)

        def comp(g, off):
            pv = as_ + g * L + iota
            v = ids_w[pl.ds(g * L, L)]
            nx = ids_w[pl.ds(g * L + 1, L)]
            keep = jnp.logical_and(
                v != nx, jnp.logical_and(pv >= pmin, pv < pe))
            ki = jnp.where(keep, ones, zeros)
            slot16 = off + plsc.cumsum(ki) - ki
            plsc.store_scatter(bid, [slot16], v, mask=keep)
            plsc.store_scatter(bpos, [slot16], pv - pbs, mask=keep)
            return off + plsc.all_reduce_population_count(keep)

        off = lax.fori_loop(0, (PB + 16) // L, comp, zeros)
        kb = off[0]

        wait_input(b)
        slv = jnp.full((L,), b & 1, jnp.int32)

        def wcond(wst):
            cj = wst[0]
            return cj < kb

        def wbody(wst):
            cj, S, slot, fl, zf = wst
            idv = bid[pl.ds(cj, L)]
            posv = bpos[pl.ds(cj, L)]
            first = idv[0]

            def do_flush(ops):
                cj, S, slot, fl, zf = ops
                S, slot, fl, zf = flush_step(S, slot, fl, zf)
                return cj, S, slot, fl, zf

            def do_place(ops):
                cj, S, slot, fl, zf = ops
                mask = jnp.logical_and(iota < (kb - cj),
                                       idv < (jnp.full((L,), 0, jnp.int32) + S + OB))
                on_slot(slot, lambda ov, fs, zs: place_16(
                    ov, slv, posv, idv, S, mask))
                cnt = plsc.all_reduce_population_count(mask)[0]
                return cj + cnt, S, slot, fl, zf

            return lax.cond(first >= S + OB, do_flush, do_place,
                            (cj, S, slot, fl, zf))

        _, S, slot, fl, zf = lax.while_loop(
            wcond, wbody, (jnp.int32(0), S, slot, fl, zf))
        return S, slot, fl, zf

    fire_input(jnp.int32(0))
    S, slot, fl, zf = lax.fori_loop(
        0, nblk, blk_body, (Tw, jnp.int32(0), jnp.int32(0), jnp.int32(2)))
    wait_input(nblk)

    nfull = (Tn - S) // OB

    def drain_full(i, st):
        S, slot, fl, zf = st
        return flush_step(S, slot, fl, zf)

    S, slot, fl, zf = lax.fori_loop(0, nfull, drain_full,
                                    (S, slot, fl, zf))

    ntail = (Tn - S) >> 7

    def drain_tail(t, c):
        def cp(ov, fs, zs):
            for g8 in range(8):
                pltpu.async_copy(ov.at[g8, pl.ds(t, 1)],
                                 o4.at[g8, pl.ds((S >> 7) + t, 1)], sem_t)

        on_slot(slot, cp)
        return c

    lax.fori_loop(0, ntail, drain_tail, 0)

    def tail_wait(t, c):
        for g8 in range(8):
            pltpu.make_async_copy(ov0.at[g8, pl.ds(0, 1)],
                                  o4.at[g8, pl.ds(0, 1)], sem_t).wait()
        return c

    lax.fori_loop(0, ntail, tail_wait, 0)

    for i, (ovx, fsx, zsx) in enumerate([(ov0, sem_0, sem_z0),
                                         (ov1, sem_1, sem_z1),
                                         (ov2, sem_2, sem_z2)]):
        @pl.when(((fl >> i) & 1) == 1)
        def _(ovx=ovx, fsx=fsx):
            wait_flush(ovx, fsx)

        @pl.when(((zf >> i) & 1) == 1)
        def _(ovx=ovx, zsx=zsx):
            wait_zero(ovx, zsx)


@jax.jit
def kernel(data, segment_ids):
    d4 = data.T.reshape(8, 8, N // 128, 128).transpose(0, 2, 1, 3)
    mesh = plsc.VectorSubcoreMesh(core_axis_name="c", subcore_axis_name="s")
    run = pl.kernel(
        _body,
        out_type=jax.ShapeDtypeStruct((8, NTO, 8, 128), jnp.float32),
        mesh=mesh,
        compiler_params=pltpu.CompilerParams(needs_layout_passes=False),
        scratch_types=[
            pltpu.VMEM((IDW + L,), jnp.int32),
            pltpu.VMEM((IDW + L,), jnp.int32),
            pltpu.VMEM((IDW + L,), jnp.int32),
            pltpu.VMEM((2, 8, TPB, 8, 128), jnp.float32),
            pltpu.VMEM((8, TOB, 8, 128), jnp.float32),
            pltpu.VMEM((8, TOB, 8, 128), jnp.float32),
            pltpu.VMEM((8, TOB, 8, 128), jnp.float32),
            pltpu.SemaphoreType.DMA,
            pltpu.SemaphoreType.DMA,
            pltpu.SemaphoreType.DMA,
            pltpu.SemaphoreType.DMA,
            pltpu.SemaphoreType.DMA,
            pltpu.SemaphoreType.DMA,
            pltpu.SemaphoreType.DMA,
            pltpu.SemaphoreType.DMA,
            pltpu.SemaphoreType.DMA,
        ],
    )
    z4 = jnp.zeros((8, TOB, 8, 128), jnp.float32)
    o4 = run(d4, segment_ids, z4)
    return o4.transpose(1, 3, 0, 2).reshape(M, C)


# speedup vs baseline: 1.6685x; 1.6685x over previous
import jax
import jax.numpy as jnp
from jax import lax
from jax.experimental import pallas as pl
from jax.experimental.pallas import tpu as pltpu
from jax.experimental.pallas import tpu_sc as plsc

N = 524288
C = 64
M = 524288
NC = 2
NS = 16
NW = NC * NS
P = N // NW
L = 16
PB = 512
OB = 384
TPB = PB // 128
TOB = OB // 128
IDW = PB + 32
NTI = N // 128
NTO = M // 128


def _body(d4, seg, o4, ids_w, bpos, bid, in_v, ov0, ov1,
          sem_i0, sem_i1, sem_0, sem_1, sem_t):
    wid = lax.axis_index("s") * NC + lax.axis_index("c")
    iota = lax.broadcasted_iota(jnp.int32, (L,), 0)
    ones = jnp.full((L,), 1, jnp.int32)
    zeros = jnp.full((L,), 0, jnp.int32)
    zf16 = jnp.zeros((L,), jnp.float32)

    pltpu.sync_copy(seg.at[pl.ds(wid * P, L)], ids_w.at[pl.ds(0, L)])

    @pl.when(wid < NW - 1)
    def _():
        pltpu.sync_copy(seg.at[pl.ds((wid + 1) * P, L)], ids_w.at[pl.ds(L, L)])

    myfirst = ids_w[pl.ds(0, L)][0]
    nxtfirst = ids_w[pl.ds(L, L)][0]
    Tw = jnp.where(wid == 0, 0, myfirst & -128)
    Tn = jnp.where(wid == NW - 1, M, nxtfirst & -128)

    def find_first_ge(Tv, anchor):
        tsplat = jnp.full((L,), 0, jnp.int32) + Tv

        def count_win(e):
            pltpu.sync_copy(seg.at[pl.ds(pl.multiple_of(e - 512, 512), 512)],
                            ids_w.at[pl.ds(0, 512)])

            def cg(g, acc):
                v = ids_w[pl.ds(g * L, L)]
                return acc + plsc.all_reduce_population_count(v >= tsplat)

            return lax.fori_loop(0, 512 // L, cg, zeros)[0]

        c0 = count_win(anchor)

        def cond(st):
            e, c = st
            return jnp.logical_and(c == 512, e > 512)

        def bdy(st):
            e, c = st
            return (e - 512, count_win(e - 512))

        eF, cF = lax.while_loop(cond, bdy, (anchor, c0))
        return eF - cF

    lo = jnp.where(wid == 0, 0,
                   find_first_ge(Tw, jnp.maximum(wid * P, 512)))
    hi = jnp.where(wid == NW - 1, N, find_first_ge(Tn, (wid + 1) * P))

    def zfill_vec(ov):
        def zz(g, c):
            for t in range(TOB):
                for c8 in range(8):
                    for q in range(8):
                        ov[g, t, c8, pl.ds(q * L, L)] = zf16
            return c

        lax.fori_loop(0, 8, zz, 0)

    def zfill(ov):
        zfill_vec(ov)

    def fire_flush(ov, sem, S):
        for g8 in range(8):
            pltpu.async_copy(ov.at[g8], o4.at[g8, pl.ds(S >> 7, TOB)], sem)

    def wait_flush(ov, sem):
        for g8 in range(8):
            pltpu.make_async_copy(ov.at[g8], o4.at[g8, pl.ds(0, TOB)],
                                  sem).wait()

    def flush_step(S, slot, fl0, fl1):
        def f0(a):
            fire_flush(ov0, sem_0, a)
            return 0

        def f1(a):
            fire_flush(ov1, sem_1, a)
            return 0

        lax.cond(slot == 0, f0, f1, S)
        nfl0 = fl0 + jnp.where(slot == 0, 1, 0)
        nfl1 = fl1 + jnp.where(slot == 0, 0, 1)

        def p1(c):
            @pl.when(nfl1 >= 1)
            def _():
                wait_flush(ov1, sem_1)

            zfill(ov1)
            return c

        def p0(c):
            @pl.when(nfl0 >= 1)
            def _():
                wait_flush(ov0, sem_0)

            zfill(ov0)
            return c

        lax.cond(slot == 0, p1, p0, 0)
        nfl0 = nfl0 - jnp.where(jnp.logical_and(slot == 1, nfl0 >= 1), 1, 0)
        nfl1 = nfl1 - jnp.where(jnp.logical_and(slot == 0, nfl1 >= 1), 1, 0)
        return S + OB, 1 - slot, nfl0, nfl1

    def binit(g, c):
        bid[pl.ds(g * L, L)] = zeros
        bpos[pl.ds(g * L, L)] = zeros
        return c

    lax.fori_loop(0, (IDW + L) // L, binit, 0)
    zfill_vec(ov0)
    pb0 = lo & -128
    nblk = jnp.where(hi > pb0, (hi - pb0 + PB - 1) // PB, 0)

    def place_16(ov, slv, posv, idv, S, mask):
        ti = posv >> 7
        li = posv & 127
        to = (idv - S) >> 7
        lo_ = (idv - S) & 127
        for g8 in range(8):
            sg = jnp.full((L,), g8, jnp.int32)
            vals = []
            for c8 in range(8):
                sc = jnp.full((L,), c8, jnp.int32)
                vals.append(
                    plsc.load_gather(in_v, [slv, sg, ti, sc, li], mask=mask))
            for c8 in range(8):
                sc = jnp.full((L,), c8, jnp.int32)
                plsc.store_scatter(ov, [sg, to, sc, lo_], vals[c8], mask=mask)

    def fire_input(b):
        pbs = jnp.minimum(pb0 + b * PB, N - PB)
        par = b & 1

        @pl.when(par == 0)
        def _():
            for g8 in range(8):
                pltpu.async_copy(d4.at[g8, pl.ds(pbs >> 7, TPB)],
                                 in_v.at[0, g8], sem_i0)

        @pl.when(par == 1)
        def _():
            for g8 in range(8):
                pltpu.async_copy(d4.at[g8, pl.ds(pbs >> 7, TPB)],
                                 in_v.at[1, g8], sem_i1)

    def wait_input(b):
        par = b & 1

        @pl.when(par == 0)
        def _():
            for g8 in range(8):
                pltpu.make_async_copy(d4.at[0, pl.ds(0, TPB)],
                                      in_v.at[0, g8], sem_i0).wait()

        @pl.when(par == 1)
        def _():
            for g8 in range(8):
                pltpu.make_async_copy(d4.at[0, pl.ds(0, TPB)],
                                      in_v.at[1, g8], sem_i1).wait()

    def blk_body(b, st):
        S, slot, fl0, fl1 = st
        pb = pb0 + b * PB
        pbs = jnp.minimum(pb, N - PB)
        pmin = jnp.maximum(pb, lo)
        pe = jnp.minimum(pb + PB, hi)

        fire_input(b + 1)

        as_ = jnp.minimum(pb, N - (PB + 16))
        pltpu.sync_copy(seg.at[pl.ds(pl.multiple_of(as_, 16), PB + 16)],
                        ids_w.at[pl.ds(0, PB + 16)])

        @pl.when(as_ == N - (PB + 16))
        def _():
            ids_w[pl.ds(PB + 16, L)] = jnp.full((L,), M, jnp.int32)

        def comp(g, off):
            pv = as_ + g * L + iota
            v = ids_w[pl.ds(g * L, L)]
            nx = ids_w[pl.ds(g * L + 1, L)]
            keep = jnp.logical_and(
                v != nx, jnp.logical_and(pv >= pmin, pv < pe))
            ki = jnp.where(keep, ones, zeros)
            slot16 = off + plsc.cumsum(ki) - ki
            plsc.store_scatter(bid, [slot16], v, mask=keep)
            plsc.store_scatter(bpos, [slot16], pv - pbs, mask=keep)
            return off + plsc.all_reduce_population_count(keep)

        off = lax.fori_loop(0, (PB + 16) // L, comp, zeros)
        kb = off[0]

        wait_input(b)
        slv = jnp.full((L,), b & 1, jnp.int32)

        def wcond(wst):
            cj = wst[0]
            return cj < kb

        def wbody(wst):
            cj, S, slot, fl0, fl1 = wst
            idv = bid[pl.ds(cj, L)]
            posv = bpos[pl.ds(cj, L)]
            first = idv[0]

            def do_flush(ops):
                cj, S, slot, fl0, fl1 = ops
                S, slot, fl0, fl1 = flush_step(S, slot, fl0, fl1)
                return cj, S, slot, fl0, fl1

            def do_place(ops):
                cj, S, slot, fl0, fl1 = ops
                lim = jnp.full((L,), 0, jnp.int32) + S + OB
                mask = jnp.logical_and(iota < (kb - cj), idv < lim)
                idv2 = bid[pl.ds(cj + L, L)]
                posv2 = bpos[pl.ds(cj + L, L)]
                mask2 = jnp.logical_and(iota < (kb - cj - L), idv2 < lim)

                def g0(c):
                    place_16(ov0, slv, posv, idv, S, mask)
                    place_16(ov0, slv, posv2, idv2, S, mask2)
                    return c

                def g1(c):
                    place_16(ov1, slv, posv, idv, S, mask)
                    place_16(ov1, slv, posv2, idv2, S, mask2)
                    return c

                lax.cond(slot == 0, g0, g1, 0)
                cnt = (plsc.all_reduce_population_count(mask)[0]
                       + plsc.all_reduce_population_count(mask2)[0])
                return cj + cnt, S, slot, fl0, fl1

            return lax.cond(first >= S + OB, do_flush, do_place,
                            (cj, S, slot, fl0, fl1))

        _, S, slot, fl0, fl1 = lax.while_loop(
            wcond, wbody, (jnp.int32(0), S, slot, fl0, fl1))
        return S, slot, fl0, fl1

    fire_input(jnp.int32(0))
    S, slot, fl0, fl1 = lax.fori_loop(
        0, nblk, blk_body, (Tw, jnp.int32(0), jnp.int32(0), jnp.int32(0)))
    wait_input(nblk)

    nfull = (Tn - S) // OB

    def drain_full(i, st):
        S, slot, fl0, fl1 = st
        return flush_step(S, slot, fl0, fl1)

    S, slot, fl0, fl1 = lax.fori_loop(0, nfull, drain_full,
                                      (S, slot, fl0, fl1))

    ntail = (Tn - S) >> 7

    def drain_tail(t, c):
        def t0(tt):
            for g8 in range(8):
                pltpu.async_copy(ov0.at[g8, pl.ds(tt, 1)],
                                 o4.at[g8, pl.ds((S >> 7) + tt, 1)], sem_t)
            return 0

        def t1(tt):
            for g8 in range(8):
                pltpu.async_copy(ov1.at[g8, pl.ds(tt, 1)],
                                 o4.at[g8, pl.ds((S >> 7) + tt, 1)], sem_t)
            return 0

        lax.cond(slot == 0, t0, t1, t)
        return c

    lax.fori_loop(0, ntail, drain_tail, 0)

    def tail_wait(t, c):
        for g8 in range(8):
            pltpu.make_async_copy(ov0.at[g8, pl.ds(0, 1)],
                                  o4.at[g8, pl.ds(0, 1)], sem_t).wait()
        return c

    lax.fori_loop(0, ntail, tail_wait, 0)

    @pl.when(fl0 >= 1)
    def _():
        wait_flush(ov0, sem_0)

    @pl.when(fl1 >= 1)
    def _():
        wait_flush(ov1, sem_1)


@jax.jit
def kernel(data, segment_ids):
    d4 = data.T.reshape(8, 8, N // 128, 128).transpose(0, 2, 1, 3)
    mesh = plsc.VectorSubcoreMesh(core_axis_name="c", subcore_axis_name="s")
    run = pl.kernel(
        _body,
        out_type=jax.ShapeDtypeStruct((8, NTO, 8, 128), jnp.float32),
        mesh=mesh,
        compiler_params=pltpu.CompilerParams(needs_layout_passes=False),
        scratch_types=[
            pltpu.VMEM((IDW + L,), jnp.int32),
            pltpu.VMEM((IDW + L,), jnp.int32),
            pltpu.VMEM((IDW + L,), jnp.int32),
            pltpu.VMEM((2, 8, TPB, 8, 128), jnp.float32),
            pltpu.VMEM((8, TOB, 8, 128), jnp.float32),
            pltpu.VMEM((8, TOB, 8, 128), jnp.float32),
            pltpu.SemaphoreType.DMA,
            pltpu.SemaphoreType.DMA,
            pltpu.SemaphoreType.DMA,
            pltpu.SemaphoreType.DMA,
            pltpu.SemaphoreType.DMA,
        ],
    )
    o4 = run(d4, segment_ids)
    return o4.transpose(1, 3, 0, 2).reshape(M, C)
